# Initial kernel scaffold; baseline (speedup 1.0000x reference)
import jax
import jax.numpy as jnp
from jax.experimental import pallas as pl


def kernel(h_est_real, h_est_imag, codebook_real, codebook_imag):
    raise NotImplementedError("write your pallas kernel here")


# speedup vs baseline: 2.3055x; 2.3055x over previous
import functools

import jax
import jax.numpy as jnp
from jax import lax
from jax.experimental import pallas as pl
from jax.experimental.pallas import tpu as pltpu
from jax.experimental.pallas import tpu_sc as plsc

_NC = 2
_NS = 16
_NW = _NC * _NS

_BT = 256
_CH = 128


def _scores_argmax_body(k_codes, x_ref, w_ref, idx_ref):
    x = x_ref[...]
    w = w_ref[...]
    c = jnp.dot(x, w, preferred_element_type=jnp.float32)
    re = c[:, :k_codes]
    im = c[:, k_codes:]
    s = re * re + im * im
    idx_ref[0, 0, :] = jnp.argmax(s, axis=1).astype(jnp.int32)


def _tc_scores_argmax(x, w, k_codes):
    b, d2 = x.shape
    nb = b // _BT
    out = pl.pallas_call(
        functools.partial(_scores_argmax_body, k_codes),
        grid=(nb,),
        in_specs=[
            pl.BlockSpec((_BT, d2), lambda i: (i, 0)),
            pl.BlockSpec((d2, 2 * k_codes), lambda i: (0, 0)),
        ],
        out_specs=pl.BlockSpec((1, 1, _BT), lambda i: (i, 0, 0)),
        out_shape=jax.ShapeDtypeStruct((nb, 1, _BT), jnp.int32),
    )(x, w)
    return out.reshape(b)


def _sc_gather(table, idx2, b, d_row):
    rows_per_w = b // _NW
    chunks = rows_per_w // _CH
    mesh = plsc.VectorSubcoreMesh(core_axis_name="c", subcore_axis_name="s")

    @functools.partial(
        pl.kernel,
        mesh=mesh,
        out_type=jax.ShapeDtypeStruct((b, d_row), jnp.float32),
        scratch_types=[
            pltpu.VMEM((chunks, _CH), jnp.int32),
            pltpu.VMEM((2 * _CH, d_row), jnp.float32),
            pltpu.SemaphoreType.DMA,
        ],
    )
    def gather_kernel(table_hbm, idx_hbm, out_hbm, idx_v, rows_v, sem):
        wid = lax.axis_index("s") * _NC + lax.axis_index("c")
        base = wid * rows_per_w
        pltpu.sync_copy(idx_hbm.at[pl.ds(wid * chunks, chunks)], idx_v)
        for t in range(chunks // 2):
            c0 = pltpu.async_copy(
                table_hbm.at[idx_v.at[2 * t]],
                rows_v.at[pl.ds(0, _CH)], sem)
            c1 = pltpu.async_copy(
                table_hbm.at[idx_v.at[2 * t + 1]],
                rows_v.at[pl.ds(_CH, _CH)], sem)
            c0.wait()
            c1.wait()
            pltpu.sync_copy(
                rows_v, out_hbm.at[pl.ds(base + t * 2 * _CH, 2 * _CH)])

    return gather_kernel(table, idx2)


def kernel(h_est_real, h_est_imag, codebook_real, codebook_imag):
    b, v, s = h_est_real.shape
    k_codes = codebook_real.shape[0]
    d = v * s

    hrf = h_est_real.reshape(b, d)
    hif = h_est_imag.reshape(b, d)
    crf = codebook_real.reshape(k_codes, d)
    cif = codebook_imag.reshape(k_codes, d)
    x = jnp.concatenate([hrf + hif, hif, hrf], axis=1)
    zero = jnp.zeros_like(crf)
    w_re = jnp.concatenate([crf, -(crf - cif), zero], axis=1)
    w_im = jnp.concatenate([crf, zero, -(crf + cif)], axis=1)
    w = jnp.concatenate([w_re, w_im], axis=0).T

    idx = _tc_scores_argmax(x, w, k_codes)

    d_row = 2 * _CH
    table = jnp.stack([codebook_real, codebook_imag], axis=-1)
    table = table.reshape(k_codes, 2 * d)
    table = jnp.pad(table, ((0, 0), (0, d_row - 2 * d)))
    idx2 = idx.reshape(b // _CH, _CH)
    rows = _sc_gather(table, idx2, b, d_row)
    return rows[:, : 2 * d].reshape(b, v, s, 2)


# speedup vs baseline: 2.9663x; 1.2866x over previous
import functools

import jax
import jax.numpy as jnp
from jax import lax
from jax.experimental import pallas as pl
from jax.experimental.pallas import tpu as pltpu
from jax.experimental.pallas import tpu_sc as plsc

_NC = 2
_NS = 16
_NW = _NC * _NS

_BT = 256
_CH = 128


def _scores_argmax_body(hr_ref, hi_ref, w1_ref, w2_ref, w3n_ref, idx_ref):
    hr = hr_ref[...]
    hi = hi_ref[...]
    pa = jnp.dot(hr + hi, w1_ref[...], preferred_element_type=jnp.float32)
    pb = jnp.dot(hi, w2_ref[...], preferred_element_type=jnp.float32)
    pc = jnp.dot(hr, w3n_ref[...], preferred_element_type=jnp.float32)
    re = pa - pb
    im = pa + pc
    s = re * re + im * im
    idx_ref[0, 0, :] = jnp.argmax(s, axis=1).astype(jnp.int32)


def _tc_scores_argmax(hr2, hi2, w1, w2, w3n):
    b, d = hr2.shape
    k_codes = w1.shape[1]
    nb = b // _BT
    wspec = pl.BlockSpec((d, k_codes), lambda i: (0, 0))
    out = pl.pallas_call(
        _scores_argmax_body,
        grid=(nb,),
        in_specs=[
            pl.BlockSpec((_BT, d), lambda i: (i, 0)),
            pl.BlockSpec((_BT, d), lambda i: (i, 0)),
            wspec, wspec, wspec,
        ],
        out_specs=pl.BlockSpec((1, 1, _BT), lambda i: (i, 0, 0)),
        out_shape=jax.ShapeDtypeStruct((nb, 1, _BT), jnp.int32),
    )(hr2, hi2, w1, w2, w3n)
    return out.reshape(b)


def _sc_gather(table, idx2, b, d_pad, d_out):
    rows_per_w = b // _NW
    chunks = rows_per_w // _CH
    mesh = plsc.VectorSubcoreMesh(core_axis_name="c", subcore_axis_name="s")

    @functools.partial(
        pl.kernel,
        mesh=mesh,
        out_type=jax.ShapeDtypeStruct((b, d_pad), jnp.float32),
        scratch_types=[
            pltpu.VMEM((chunks, _CH), jnp.int32),
            pltpu.VMEM((2 * _CH, d_pad), jnp.float32),
            pltpu.SemaphoreType.DMA,
        ],
    )
    def gather_kernel(table_hbm, idx_hbm, out_hbm, idx_v, rows_v, sem):
        wid = lax.axis_index("s") * _NC + lax.axis_index("c")
        base = wid * rows_per_w
        pltpu.sync_copy(idx_hbm.at[pl.ds(wid * chunks, chunks)], idx_v)
        for t in range(chunks // 2):
            c0 = pltpu.async_copy(
                table_hbm.at[idx_v.at[2 * t]],
                rows_v.at[pl.ds(0, _CH)], sem)
            c1 = pltpu.async_copy(
                table_hbm.at[idx_v.at[2 * t + 1]],
                rows_v.at[pl.ds(_CH, _CH)], sem)
            c0.wait()
            c1.wait()
            pltpu.sync_copy(
                rows_v, out_hbm.at[pl.ds(base + t * 2 * _CH, 2 * _CH)])

    return gather_kernel(table, idx2)


def kernel(h_est_real, h_est_imag, codebook_real, codebook_imag):
    b, v, s = h_est_real.shape
    k_codes = codebook_real.shape[0]
    d = v * s

    hr2 = h_est_real.reshape(b, d)
    hi2 = h_est_imag.reshape(b, d)
    crf = codebook_real.reshape(k_codes, d)
    cif = codebook_imag.reshape(k_codes, d)
    w1 = crf.T
    w2 = (crf - cif).T
    w3n = (-(crf + cif)).T

    idx = _tc_scores_argmax(hr2, hi2, w1, w2, w3n)

    d_pad = 2 * _CH
    table = jnp.stack([codebook_real, codebook_imag], axis=-1)
    table = table.reshape(k_codes, 2 * d)
    table = jnp.pad(table, ((0, 0), (0, d_pad - 2 * d)))
    idx2 = idx.reshape(b // _CH, _CH)
    rows = _sc_gather(table, idx2, b, d_pad, 2 * d)
    return rows[:, : 2 * d].reshape(b, v, s, 2)


# speedup vs baseline: 4.0904x; 1.3790x over previous
import functools

import jax
import jax.numpy as jnp
from jax import lax
from jax.experimental import pallas as pl
from jax.experimental.pallas import tpu as pltpu
from jax.experimental.pallas import tpu_sc as plsc

_NC = 2
_NS = 16
_NW = _NC * _NS

_BT = 256
_CH = 128


def _scores_argmax_body(hr_ref, hi_ref, w1_ref, w2_ref, w3n_ref, idx_ref):
    hr = hr_ref[...]
    hi = hi_ref[...]
    d = hr.shape[0] * hr.shape[1]
    hrt = hr.reshape(d, hr.shape[2])
    hit = hi.reshape(d, hi.shape[2])
    pa = jnp.dot(w1_ref[...], hrt + hit, preferred_element_type=jnp.float32)
    pb = jnp.dot(w2_ref[...], hit, preferred_element_type=jnp.float32)
    pc = jnp.dot(w3n_ref[...], hrt, preferred_element_type=jnp.float32)
    re = pa - pb
    im = pa + pc
    s = re * re + im * im
    idx_ref[0, 0, :] = jnp.argmax(s, axis=0).astype(jnp.int32)


def _tc_scores_argmax(hrt, hit, w1, w2, w3n):
    s_dim, v_dim, b = hrt.shape
    k_codes = w1.shape[0]
    d = s_dim * v_dim
    nb = b // _BT
    wspec = pl.BlockSpec((k_codes, d), lambda i: (0, 0))
    out = pl.pallas_call(
        _scores_argmax_body,
        grid=(nb,),
        in_specs=[
            pl.BlockSpec((s_dim, v_dim, _BT), lambda i: (0, 0, i)),
            pl.BlockSpec((s_dim, v_dim, _BT), lambda i: (0, 0, i)),
            wspec, wspec, wspec,
        ],
        out_specs=pl.BlockSpec((1, 1, _BT), lambda i: (i, 0, 0)),
        out_shape=jax.ShapeDtypeStruct((nb, 1, _BT), jnp.int32),
    )(hrt, hit, w1, w2, w3n)
    return out.reshape(b)


def _sc_gather(table, idx2, b, d_pad, d_out):
    rows_per_w = b // _NW
    chunks = rows_per_w // _CH
    mesh = plsc.VectorSubcoreMesh(core_axis_name="c", subcore_axis_name="s")

    @functools.partial(
        pl.kernel,
        mesh=mesh,
        out_type=jax.ShapeDtypeStruct((b, d_pad), jnp.float32),
        scratch_types=[
            pltpu.VMEM((chunks, _CH), jnp.int32),
            pltpu.VMEM((2 * _CH, d_pad), jnp.float32),
            pltpu.SemaphoreType.DMA,
        ],
    )
    def gather_kernel(table_hbm, idx_hbm, out_hbm, idx_v, rows_v, sem):
        wid = lax.axis_index("s") * _NC + lax.axis_index("c")
        base = wid * rows_per_w
        pltpu.sync_copy(idx_hbm.at[pl.ds(wid * chunks, chunks)], idx_v)
        for t in range(chunks // 2):
            c0 = pltpu.async_copy(
                table_hbm.at[idx_v.at[2 * t]],
                rows_v.at[pl.ds(0, _CH)], sem)
            c1 = pltpu.async_copy(
                table_hbm.at[idx_v.at[2 * t + 1]],
                rows_v.at[pl.ds(_CH, _CH)], sem)
            c0.wait()
            c1.wait()
            pltpu.sync_copy(
                rows_v, out_hbm.at[pl.ds(base + t * 2 * _CH, 2 * _CH)])

    return gather_kernel(table, idx2)


def kernel(h_est_real, h_est_imag, codebook_real, codebook_imag):
    b, v, s = h_est_real.shape
    k_codes = codebook_real.shape[0]
    d = v * s

    hrt = h_est_real.transpose(2, 1, 0)
    hit = h_est_imag.transpose(2, 1, 0)
    crf = codebook_real.transpose(0, 2, 1).reshape(k_codes, d)
    cif = codebook_imag.transpose(0, 2, 1).reshape(k_codes, d)
    w1 = crf
    w2 = crf - cif
    w3n = -(crf + cif)

    idx = _tc_scores_argmax(hrt, hit, w1, w2, w3n)

    d_pad = 2 * _CH
    table = jnp.stack([codebook_real, codebook_imag], axis=-1)
    table = table.reshape(k_codes, 2 * d)
    table = jnp.pad(table, ((0, 0), (0, d_pad - 2 * d)))
    idx2 = idx.reshape(b // _CH, _CH)
    rows = _sc_gather(table, idx2, b, d_pad, 2 * d)
    return rows[:, : 2 * d].reshape(b, v, s, 2)
